# baseline (speedup 1.0000x reference)
import functools

import jax
import jax.numpy as jnp
from jax import lax
from jax.experimental import pallas as pl
from jax.experimental.pallas import tpu as pltpu
from jax.experimental.pallas import tpu_sc as plsc

B = 4
N = 4096
D = 2048
PROB_KEEP = 0.9
K = max(1, int(N * PROB_KEEP))
TOTAL = B * K
NC = 2
NS = 16
W = 16
WPB = K // W
TAIL = K - WPB * W
TPB = (NC * NS) // B
WPT = -(-WPB // TPB)

VREGS = N // 16

_mesh = plsc.VectorSubcoreMesh(
    core_axis_name="c", subcore_axis_name="s", num_cores=NC, num_subcores=NS
)


@functools.partial(
    pl.kernel,
    out_type=jax.ShapeDtypeStruct((K, D // 128, B * 128), jnp.float32),
    mesh=_mesh,
    compiler_params=pltpu.CompilerParams(needs_layout_passes=False),
    scratch_types=[
        pltpu.VMEM((N,), jnp.int32),
        pltpu.VMEM((N,), jnp.int32),
        pltpu.VMEM((N,), jnp.int32),
        pltpu.VMEM((N,), jnp.int32),
        pltpu.VMEM((N,), jnp.int32),
        pltpu.VMEM((N,), jnp.int32),
        pltpu.VMEM_SHARED((B * N,), jnp.int32),
        pltpu.VMEM((N,), jnp.int32),
        pltpu.VMEM((16,), jnp.int32),
        pltpu.VMEM((16,), jnp.int32),
        pltpu.VMEM((W, D), jnp.float32),
        pltpu.VMEM((W, D), jnp.float32),
        pltpu.SemaphoreType.DMA,
        pltpu.SemaphoreType.DMA,
        pltpu.SemaphoreType.DMA,
        pltpu.SemaphoreType.DMA,
    ],
)
def _dropout_kernel(x_hbm, noise_hbm, out_hbm,
                    noise_v, key_a, key_b, idx_a, idx_b, cnt, tab_sh, table_v,
                    idx0, idx1, buf0, buf1,
                    sem_in0, sem_in1, sem_out0, sem_out1):
    cid = lax.axis_index("c")
    sid = lax.axis_index("s")
    lane = lax.iota(jnp.int32, 16)

    @pl.when(sid < B)
    def _():
        bsel = sid
        pltpu.sync_copy(noise_hbm.at[pl.ds(bsel * N, N)], noise_v)
        ones = jnp.ones((16,), jnp.int32)

        def init_body(j, carry):
            s = noise_v[pl.ds(j * 16, 16)]
            s = jnp.where(s == jnp.int32(-2147483648), jnp.int32(0), s)
            negm = lax.shift_right_arithmetic(s, 31)
            key = s ^ (jnp.bitwise_not(negm) & jnp.int32(0x7FFFFFFF))
            e = j * 16 + lane
            m = ((e & 255) << 4) | lax.shift_right_logical(e, 8)
            plsc.store_scatter(key_a, [m], key)
            plsc.store_scatter(idx_a, [m], e)
            return carry

        lax.fori_loop(0, VREGS, init_body, 0)

        bufs = [(key_a, idx_a), (key_b, idx_b)]
        for p in range(4):
            src_k, src_i = bufs[p % 2]
            dst_k, dst_i = bufs[(p + 1) % 2]
            shift = 8 * p

            def zero_body(j, carry):
                cnt[pl.ds(j * 16, 16)] = jnp.zeros((16,), jnp.int32)
                return carry

            lax.fori_loop(0, VREGS, zero_body, 0)

            def hist_body(j, carry, src_k=src_k, shift=shift):
                kv = src_k[pl.ds(j * 16, 16)]
                d = lax.shift_right_logical(kv, shift) & 255
                plsc.addupdate_scatter(cnt, [(d << 4) | lane], ones)
                return carry

            lax.fori_loop(0, VREGS, hist_body, 0)

            def scan_body(j, carry):
                v = cnt[pl.ds(j * 16, 16)]
                cs = plsc.cumsum(v)
                cnt[pl.ds(j * 16, 16)] = cs - v + carry
                return carry + jnp.sum(v)

            lax.fori_loop(0, VREGS, scan_body, jnp.int32(0))

            last = p == 3

            def perm_body(j, carry, src_k=src_k, src_i=src_i, dst_k=dst_k,
                          dst_i=dst_i, shift=shift, last=last):
                kv = src_k[pl.ds(j * 16, 16)]
                iv = src_i[pl.ds(j * 16, 16)]
                d = lax.shift_right_logical(kv, shift) & 255
                c = (d << 4) | lane
                pos = plsc.load_gather(cnt, [c])
                plsc.store_scatter(cnt, [c], pos + 1)
                if last:
                    m2 = pos
                else:
                    m2 = ((pos & 255) << 4) | lax.shift_right_logical(pos, 8)
                    plsc.store_scatter(dst_k, [m2], kv)
                plsc.store_scatter(dst_i, [m2], iv)
                return carry

            lax.fori_loop(0, VREGS, perm_body, 0)

        pltpu.sync_copy(idx_a, tab_sh.at[pl.ds(bsel * N, N)])

    plsc.subcore_barrier()

    gid = sid * NC + cid
    bsel = lax.shift_right_logical(gid, 3)
    t8 = gid & 7
    pltpu.sync_copy(tab_sh.at[pl.ds(bsel * N, N)], table_v)
    lo = t8 * WPT
    hi = jnp.minimum(lo + WPT, WPB)
    nt = hi - lo
    x_b = x_hbm.at[bsel]
    cofs = bsel * 128

    idx_slots = [idx0, idx1]
    buf_slots = [buf0, buf1]
    sem_in = [sem_in0, sem_in1]
    sem_out = [sem_out0, sem_out1]

    def fetch_ids(start, slot):
        g = plsc.load_gather(table_v, [start + lane])
        idx_slots[slot][...] = g

    def wait_out(slot):
        for dblk in range(D // 128):
            pltpu.make_async_copy(buf_slots[slot].at[:, pl.ds(0, 128)],
                                  out_hbm.at[pl.ds(0, W), 0, pl.ds(0, 128)],
                                  sem_out[slot]).wait()

    def do_window(w, slot):
        @pl.when(w - lo >= 2)
        def _():
            wait_out(slot)

        start = w * W
        fetch_ids(start, slot)
        pltpu.async_copy(x_b.at[idx_slots[slot]], buf_slots[slot],
                         sem_in[slot]).wait()
        for dblk in range(D // 128):
            pltpu.async_copy(
                buf_slots[slot].at[:, pl.ds(dblk * 128, 128)],
                out_hbm.at[pl.ds(start, W), dblk, pl.ds(cofs, 128)],
                sem_out[slot])

    def pair_body(q, carry):
        w0 = lo + 2 * q
        for s in range(2):
            w = w0 + s

            @pl.when(w < hi)
            def _(w=w, s=s):
                do_window(w, s)

        return carry

    lax.fori_loop(0, (WPT + 1) // 2, pair_body, 0)

    @pl.when(nt >= 2)
    def _():
        wait_out(0)
        wait_out(1)

    @pl.when(nt == 1)
    def _():
        wait_out(0)

    @pl.when(t8 == 7)
    def _():
        fetch_ids(WPB * W, 0)
        pltpu.async_copy(x_b.at[idx0], buf0, sem_in[0]).wait()
        for dblk in range(D // 128):
            pltpu.sync_copy(
                buf0.at[pl.ds(0, TAIL), pl.ds(dblk * 128, 128)],
                out_hbm.at[pl.ds(WPB * W, TAIL), dblk, pl.ds(cofs, 128)])


def kernel(x, noise):
    b, n, d = x.shape
    assert (b, n, d) == (B, N, D)
    noise_bits = lax.bitcast_convert_type(noise.reshape(B * N), jnp.int32)
    out_kdb = _dropout_kernel(x, noise_bits)
    return (out_kdb.reshape(K, D // 128, B, 128)
            .transpose(2, 0, 1, 3)
            .reshape(B, K, D))


# speedup vs baseline: 1.2145x; 1.2145x over previous
import functools

import jax
import jax.numpy as jnp
from jax import lax
from jax.experimental import pallas as pl
from jax.experimental.pallas import tpu as pltpu
from jax.experimental.pallas import tpu_sc as plsc

B = 4
N = 4096
D = 2048
PROB_KEEP = 0.9
K = max(1, int(N * PROB_KEEP))
TOTAL = B * K
NC = 2
NS = 16
W = 16
WPB = K // W
TAIL = K - WPB * W
TPB = (NC * NS) // B
WPT = -(-WPB // TPB)

VREGS = N // 16

_mesh = plsc.VectorSubcoreMesh(
    core_axis_name="c", subcore_axis_name="s", num_cores=NC, num_subcores=NS
)


@functools.partial(
    pl.kernel,
    out_type=jax.ShapeDtypeStruct((B, K, D), jnp.float32),
    mesh=_mesh,
    compiler_params=pltpu.CompilerParams(needs_layout_passes=False),
    scratch_types=[
        pltpu.VMEM((N,), jnp.int32),
        pltpu.VMEM((N,), jnp.int32),
        pltpu.VMEM((N,), jnp.int32),
        pltpu.VMEM((N,), jnp.int32),
        pltpu.VMEM((N,), jnp.int32),
        pltpu.VMEM((N,), jnp.int32),
        pltpu.VMEM_SHARED((B * N,), jnp.int32),
        pltpu.VMEM((N,), jnp.int32),
        pltpu.VMEM((16,), jnp.int32),
        pltpu.VMEM((16,), jnp.int32),
        pltpu.VMEM((W, D), jnp.float32),
        pltpu.VMEM((W, D), jnp.float32),
        pltpu.SemaphoreType.DMA,
        pltpu.SemaphoreType.DMA,
        pltpu.SemaphoreType.DMA,
        pltpu.SemaphoreType.DMA,
    ],
)
def _dropout_kernel(x_hbm, noise_hbm, out_hbm,
                    noise_v, key_a, key_b, idx_a, idx_b, cnt, tab_sh, table_v,
                    idx0, idx1, buf0, buf1,
                    sem_in0, sem_in1, sem_out0, sem_out1):
    cid = lax.axis_index("c")
    sid = lax.axis_index("s")
    lane = lax.iota(jnp.int32, 16)

    @pl.when(sid < B)
    def _():
        bsel = sid
        pltpu.sync_copy(noise_hbm.at[pl.ds(bsel * N, N)], noise_v)
        ones = jnp.ones((16,), jnp.int32)

        def init_body(j, carry):
            s = noise_v[pl.ds(j * 16, 16)]
            s = jnp.where(s == jnp.int32(-2147483648), jnp.int32(0), s)
            negm = lax.shift_right_arithmetic(s, 31)
            key = s ^ (jnp.bitwise_not(negm) & jnp.int32(0x7FFFFFFF))
            e = j * 16 + lane
            m = ((e & 255) << 4) | lax.shift_right_logical(e, 8)
            plsc.store_scatter(key_a, [m], key)
            plsc.store_scatter(idx_a, [m], e)
            return carry

        lax.fori_loop(0, VREGS, init_body, 0)

        bufs = [(key_a, idx_a), (key_b, idx_b)]
        for p in range(4):
            src_k, src_i = bufs[p % 2]
            dst_k, dst_i = bufs[(p + 1) % 2]
            shift = 8 * p

            def zero_body(j, carry):
                cnt[pl.ds(j * 16, 16)] = jnp.zeros((16,), jnp.int32)
                return carry

            lax.fori_loop(0, VREGS, zero_body, 0)

            def hist_body(j, carry, src_k=src_k, shift=shift):
                kv = src_k[pl.ds(j * 16, 16)]
                d = lax.shift_right_logical(kv, shift) & 255
                plsc.addupdate_scatter(cnt, [(d << 4) | lane], ones)
                return carry

            lax.fori_loop(0, VREGS, hist_body, 0)

            def scan_body(j, carry):
                v = cnt[pl.ds(j * 16, 16)]
                cs = plsc.cumsum(v)
                cnt[pl.ds(j * 16, 16)] = cs - v + carry
                return carry + jnp.sum(v)

            lax.fori_loop(0, VREGS, scan_body, jnp.int32(0))

            last = p == 3

            def perm_body(j, carry, src_k=src_k, src_i=src_i, dst_k=dst_k,
                          dst_i=dst_i, shift=shift, last=last):
                kv = src_k[pl.ds(j * 16, 16)]
                iv = src_i[pl.ds(j * 16, 16)]
                d = lax.shift_right_logical(kv, shift) & 255
                c = (d << 4) | lane
                pos = plsc.load_gather(cnt, [c])
                plsc.store_scatter(cnt, [c], pos + 1)
                if last:
                    m2 = pos
                else:
                    m2 = ((pos & 255) << 4) | lax.shift_right_logical(pos, 8)
                    plsc.store_scatter(dst_k, [m2], kv)
                plsc.store_scatter(dst_i, [m2], iv)
                return carry

            lax.fori_loop(0, VREGS, perm_body, 0)

        pltpu.sync_copy(idx_a, tab_sh.at[pl.ds(bsel * N, N)])

    plsc.subcore_barrier()

    gid = sid * NC + cid
    bsel = lax.shift_right_logical(gid, 3)
    t8 = gid & 7
    pltpu.sync_copy(tab_sh.at[pl.ds(bsel * N, N)], table_v)
    lo = t8 * WPT
    hi = jnp.minimum(lo + WPT, WPB)
    nt = hi - lo
    x_b = x_hbm.at[bsel]
    out_b = out_hbm.at[bsel]
    tab_base = 0

    idx_slots = [idx0, idx1]
    buf_slots = [buf0, buf1]
    sem_in = [sem_in0, sem_in1]
    sem_out = [sem_out0, sem_out1]

    def fetch_ids(start, slot):
        g = plsc.load_gather(table_v, [start + lane])
        idx_slots[slot][...] = g

    def wait_out(slot):
        pltpu.make_async_copy(buf_slots[slot], out_b.at[pl.ds(0, W)],
                              sem_out[slot]).wait()

    def do_window(w, slot):
        @pl.when(w - lo >= 2)
        def _():
            wait_out(slot)

        start = w * W
        fetch_ids(start, slot)
        pltpu.async_copy(x_b.at[idx_slots[slot]], buf_slots[slot],
                         sem_in[slot]).wait()
        pltpu.async_copy(buf_slots[slot], out_b.at[pl.ds(start, W)],
                         sem_out[slot])

    def pair_body(q, carry):
        w0 = lo + 2 * q
        for s in range(2):
            w = w0 + s

            @pl.when(w < hi)
            def _(w=w, s=s):
                do_window(w, s)

        return carry

    lax.fori_loop(0, (WPT + 1) // 2, pair_body, 0)

    @pl.when(nt >= 2)
    def _():
        wait_out(0)
        wait_out(1)

    @pl.when(nt == 1)
    def _():
        wait_out(0)

    @pl.when(t8 == 7)
    def _():
        fetch_ids(WPB * W, 0)
        pltpu.async_copy(x_b.at[idx0], buf0, sem_in[0]).wait()
        pltpu.sync_copy(buf0.at[pl.ds(0, TAIL)],
                        out_b.at[pl.ds(WPB * W, TAIL)])


def kernel(x, noise):
    b, n, d = x.shape
    assert (b, n, d) == (B, N, D)
    noise_bits = lax.bitcast_convert_type(noise.reshape(B * N), jnp.int32)
    out = _dropout_kernel(x, noise_bits)
    return lax.optimization_barrier(out)


# speedup vs baseline: 1.2166x; 1.0017x over previous
import functools

import jax
import jax.numpy as jnp
from jax import lax
from jax.experimental import pallas as pl
from jax.experimental.pallas import tpu as pltpu
from jax.experimental.pallas import tpu_sc as plsc

B = 4
N = 4096
D = 2048
PROB_KEEP = 0.9
K = max(1, int(N * PROB_KEEP))
TOTAL = B * K
NC = 2
NS = 16
W = 16
WPB = K // W
TAIL = K - WPB * W
TPB = (NC * NS) // B
WPT = -(-WPB // TPB)

VREGS = N // 16

_mesh = plsc.VectorSubcoreMesh(
    core_axis_name="c", subcore_axis_name="s", num_cores=NC, num_subcores=NS
)


@functools.partial(
    pl.kernel,
    out_type=jax.ShapeDtypeStruct((B, K, D), jnp.float32),
    mesh=_mesh,
    compiler_params=pltpu.CompilerParams(needs_layout_passes=False),
    scratch_types=[
        pltpu.VMEM((N,), jnp.int32),
        pltpu.VMEM((N,), jnp.int32),
        pltpu.VMEM((N,), jnp.int32),
        pltpu.VMEM((N,), jnp.int32),
        pltpu.VMEM((N,), jnp.int32),
        pltpu.VMEM((N,), jnp.int32),
        pltpu.VMEM_SHARED((B * N,), jnp.int32),
        pltpu.VMEM((N,), jnp.int32),
        pltpu.VMEM((16,), jnp.int32),
        pltpu.VMEM((16,), jnp.int32),
        pltpu.VMEM((W, D), jnp.float32),
        pltpu.VMEM((W, D), jnp.float32),
        pltpu.SemaphoreType.DMA,
        pltpu.SemaphoreType.DMA,
        pltpu.SemaphoreType.DMA,
        pltpu.SemaphoreType.DMA,
    ],
)
def _dropout_kernel(x_hbm, noise_hbm, out_hbm,
                    noise_v, key_a, key_b, idx_a, idx_b, cnt, tab_sh, table_v,
                    idx0, idx1, buf0, buf1,
                    sem_in0, sem_in1, sem_out0, sem_out1):
    cid = lax.axis_index("c")
    sid = lax.axis_index("s")
    lane = lax.iota(jnp.int32, 16)

    @pl.when(sid < B)
    def _():
        bsel = sid
        pltpu.sync_copy(noise_hbm.at[pl.ds(bsel * N, N)], noise_v)
        ones = jnp.ones((16,), jnp.int32)

        def init_body(j, carry):
            s = noise_v[pl.ds(j * 16, 16)]
            s = jnp.where(s == jnp.int32(-2147483648), jnp.int32(0), s)
            negm = lax.shift_right_arithmetic(s, 31)
            key = s ^ (jnp.bitwise_not(negm) & jnp.int32(0x7FFFFFFF))
            e = j * 16 + lane
            m = ((e & 255) << 4) | lax.shift_right_logical(e, 8)
            plsc.store_scatter(key_a, [m], key)
            plsc.store_scatter(idx_a, [m], e)
            return carry

        lax.fori_loop(0, VREGS, init_body, 0)

        bufs = [(key_a, idx_a), (key_b, idx_b)]
        for p in range(4):
            src_k, src_i = bufs[p % 2]
            dst_k, dst_i = bufs[(p + 1) % 2]
            shift = 8 * p

            def zero_body(j, carry):
                cnt[pl.ds(j * 16, 16)] = jnp.zeros((16,), jnp.int32)
                return carry

            lax.fori_loop(0, VREGS, zero_body, 0)

            def hist_body(j, carry, src_k=src_k, shift=shift):
                kv = src_k[pl.ds(j * 16, 16)]
                d = lax.shift_right_logical(kv, shift) & 255
                plsc.addupdate_scatter(cnt, [(d << 4) | lane], ones)
                return carry

            lax.fori_loop(0, VREGS, hist_body, 0)

            def scan_body(j, carry):
                v = cnt[pl.ds(j * 16, 16)]
                cs = plsc.cumsum(v)
                cnt[pl.ds(j * 16, 16)] = cs - v + carry
                return carry + jnp.sum(v)

            lax.fori_loop(0, VREGS, scan_body, jnp.int32(0))

            last = p == 3

            def perm_body(j, carry, src_k=src_k, src_i=src_i, dst_k=dst_k,
                          dst_i=dst_i, shift=shift, last=last):
                kv = src_k[pl.ds(j * 16, 16)]
                iv = src_i[pl.ds(j * 16, 16)]
                d = lax.shift_right_logical(kv, shift) & 255
                c = (d << 4) | lane
                pos = plsc.load_gather(cnt, [c])
                plsc.store_scatter(cnt, [c], pos + 1)
                if last:
                    m2 = pos
                else:
                    m2 = ((pos & 255) << 4) | lax.shift_right_logical(pos, 8)
                    plsc.store_scatter(dst_k, [m2], kv)
                plsc.store_scatter(dst_i, [m2], iv)
                return carry

            lax.fori_loop(0, VREGS, perm_body, 0)

        pltpu.sync_copy(idx_a, tab_sh.at[pl.ds(bsel * N, N)])

    plsc.subcore_barrier()

    gid = sid * NC + cid
    bsel = lax.shift_right_logical(gid, 3)
    t8 = gid & 7
    pltpu.sync_copy(tab_sh.at[pl.ds(bsel * N, N)], table_v)
    lo = t8 * WPT
    hi = jnp.minimum(lo + WPT, WPB)
    nt = hi - lo
    x_b = x_hbm.at[bsel]
    out_b = out_hbm.at[bsel]
    tab_base = 0

    idx_slots = [idx0, idx1]
    buf_slots = [buf0, buf1]
    sem_in = [sem_in0, sem_in1]
    sem_out = [sem_out0, sem_out1]

    def fetch_ids(start, slot):
        g = plsc.load_gather(table_v, [start + lane])
        idx_slots[slot][...] = g

    def wait_out(slot):
        pltpu.make_async_copy(buf_slots[slot], out_b.at[pl.ds(0, W)],
                              sem_out[slot]).wait()

    def start_in(w, slot):
        fetch_ids(w * W, slot)
        pltpu.async_copy(x_b.at[idx_slots[slot]], buf_slots[slot],
                         sem_in[slot])

    def wait_in(slot):
        pltpu.make_async_copy(x_b.at[idx_slots[slot]], buf_slots[slot],
                              sem_in[slot]).wait()

    def do_window(w, slot):
        wait_in(slot)

        @pl.when(w + 1 < hi)
        def _():
            @pl.when(w - lo >= 1)
            def _():
                wait_out(1 - slot)

            start_in(w + 1, 1 - slot)

        pltpu.async_copy(buf_slots[slot], out_b.at[pl.ds(w * W, W)],
                         sem_out[slot])

    def pair_body(q, carry):
        w0 = lo + 2 * q
        for s in range(2):
            w = w0 + s

            @pl.when(w < hi)
            def _(w=w, s=s):
                do_window(w, s)

        return carry

    @pl.when(nt >= 1)
    def _():
        start_in(lo, 0)

    lax.fori_loop(0, (WPT + 1) // 2, pair_body, 0)

    @pl.when(nt >= 2)
    def _():
        wait_out(0)
        wait_out(1)

    @pl.when(nt == 1)
    def _():
        wait_out(0)

    @pl.when(t8 == 7)
    def _():
        fetch_ids(WPB * W, 0)
        pltpu.async_copy(x_b.at[idx0], buf0, sem_in[0]).wait()
        pltpu.sync_copy(buf0.at[pl.ds(0, TAIL)],
                        out_b.at[pl.ds(WPB * W, TAIL)])


def kernel(x, noise):
    b, n, d = x.shape
    assert (b, n, d) == (B, N, D)
    noise_bits = lax.bitcast_convert_type(noise.reshape(B * N), jnp.int32)
    out = _dropout_kernel(x, noise_bits)
    return lax.optimization_barrier(out)


# speedup vs baseline: 1.9032x; 1.5644x over previous
import functools

import jax
import jax.numpy as jnp
from jax import lax
from jax.experimental import pallas as pl
from jax.experimental.pallas import tpu as pltpu
from jax.experimental.pallas import tpu_sc as plsc

B = 4
N = 4096
D = 2048
PROB_KEEP = 0.9
K = max(1, int(N * PROB_KEEP))
TOTAL = B * K
NC = 2
NS = 16
W = 16
WPB = K // W
TAIL = K - WPB * W
TPB = (NC * NS) // B
WPT = -(-WPB // TPB)

VREGS = N // 16

_mesh = plsc.VectorSubcoreMesh(
    core_axis_name="c", subcore_axis_name="s", num_cores=NC, num_subcores=NS
)


@functools.partial(
    pl.kernel,
    out_type=jax.ShapeDtypeStruct((K, D // 128, B, 128), jnp.float32),
    mesh=_mesh,
    compiler_params=pltpu.CompilerParams(needs_layout_passes=False),
    scratch_types=[
        pltpu.VMEM((N,), jnp.int32),
        pltpu.VMEM((N,), jnp.int32),
        pltpu.VMEM((N,), jnp.int32),
        pltpu.VMEM((N,), jnp.int32),
        pltpu.VMEM((N,), jnp.int32),
        pltpu.VMEM((N,), jnp.int32),
        pltpu.VMEM_SHARED((B * N,), jnp.int32),
        pltpu.VMEM((N,), jnp.int32),
        pltpu.VMEM((16,), jnp.int32),
        pltpu.VMEM((16,), jnp.int32),
        pltpu.VMEM((W, D), jnp.float32),
        pltpu.VMEM((W, D), jnp.float32),
        pltpu.SemaphoreType.DMA,
        pltpu.SemaphoreType.DMA,
        pltpu.SemaphoreType.DMA,
        pltpu.SemaphoreType.DMA,
    ],
)
def _dropout_kernel(x_hbm, noise_hbm, out_hbm,
                    noise_v, key_a, key_b, idx_a, idx_b, cnt, tab_sh, table_v,
                    idx0, idx1, buf0, buf1,
                    sem_in0, sem_in1, sem_out0, sem_out1):
    cid = lax.axis_index("c")
    sid = lax.axis_index("s")
    lane = lax.iota(jnp.int32, 16)

    @pl.when(sid < B)
    def _():
        bsel = sid
        pltpu.sync_copy(noise_hbm.at[pl.ds(bsel * N, N)], noise_v)
        ones = jnp.ones((16,), jnp.int32)

        def init_body(j, carry):
            s = noise_v[pl.ds(j * 16, 16)]
            s = jnp.where(s == jnp.int32(-2147483648), jnp.int32(0), s)
            negm = lax.shift_right_arithmetic(s, 31)
            key = s ^ (jnp.bitwise_not(negm) & jnp.int32(0x7FFFFFFF))
            e = j * 16 + lane
            m = ((e & 255) << 4) | lax.shift_right_logical(e, 8)
            plsc.store_scatter(key_a, [m], key)
            plsc.store_scatter(idx_a, [m], e)
            return carry

        lax.fori_loop(0, VREGS, init_body, 0)

        bufs = [(key_a, idx_a), (key_b, idx_b)]
        for p in range(4):
            src_k, src_i = bufs[p % 2]
            dst_k, dst_i = bufs[(p + 1) % 2]
            shift = 8 * p

            def zero_body(j, carry):
                cnt[pl.ds(j * 16, 16)] = jnp.zeros((16,), jnp.int32)
                return carry

            lax.fori_loop(0, VREGS, zero_body, 0)

            def hist_body(j, carry, src_k=src_k, shift=shift):
                kv = src_k[pl.ds(j * 16, 16)]
                d = lax.shift_right_logical(kv, shift) & 255
                plsc.addupdate_scatter(cnt, [(d << 4) | lane], ones)
                return carry

            lax.fori_loop(0, VREGS, hist_body, 0)

            def scan_body(j, carry):
                v = cnt[pl.ds(j * 16, 16)]
                cs = plsc.cumsum(v)
                cnt[pl.ds(j * 16, 16)] = cs - v + carry
                return carry + jnp.sum(v)

            lax.fori_loop(0, VREGS, scan_body, jnp.int32(0))

            last = p == 3

            def perm_body(j, carry, src_k=src_k, src_i=src_i, dst_k=dst_k,
                          dst_i=dst_i, shift=shift, last=last):
                kv = src_k[pl.ds(j * 16, 16)]
                iv = src_i[pl.ds(j * 16, 16)]
                d = lax.shift_right_logical(kv, shift) & 255
                c = (d << 4) | lane
                pos = plsc.load_gather(cnt, [c])
                plsc.store_scatter(cnt, [c], pos + 1)
                if last:
                    m2 = pos
                else:
                    m2 = ((pos & 255) << 4) | lax.shift_right_logical(pos, 8)
                    plsc.store_scatter(dst_k, [m2], kv)
                plsc.store_scatter(dst_i, [m2], iv)
                return carry

            lax.fori_loop(0, VREGS, perm_body, 0)

        pltpu.sync_copy(idx_a, tab_sh.at[pl.ds(bsel * N, N)])

    plsc.subcore_barrier()

    gid = sid * NC + cid
    bsel = lax.shift_right_logical(gid, 3)
    t8 = gid & 7
    pltpu.sync_copy(tab_sh.at[pl.ds(bsel * N, N)], table_v)
    lo = t8 * WPT
    hi = jnp.minimum(lo + WPT, WPB)
    nt = hi - lo
    x_b = x_hbm.at[bsel]

    idx_slots = [idx0, idx1]
    buf_slots = [buf0, buf1]
    sem_in = [sem_in0, sem_in1]
    sem_out = [sem_out0, sem_out1]

    def fetch_ids(start, slot):
        g = plsc.load_gather(table_v, [start + lane])
        idx_slots[slot][...] = g

    def wait_out(slot):
        for dblk in range(D // 128):
            pltpu.make_async_copy(buf_slots[slot].at[:, pl.ds(0, 128)],
                                  out_hbm.at[pl.ds(0, W), 0, 0, :],
                                  sem_out[slot]).wait()

    def start_in(w, slot):
        fetch_ids(w * W, slot)
        pltpu.async_copy(x_b.at[idx_slots[slot]], buf_slots[slot],
                         sem_in[slot])

    def wait_in(slot):
        pltpu.make_async_copy(x_b.at[idx_slots[slot]], buf_slots[slot],
                              sem_in[slot]).wait()

    def do_window(w, slot):
        wait_in(slot)

        @pl.when(w + 1 < hi)
        def _():
            @pl.when(w - lo >= 1)
            def _():
                wait_out(1 - slot)

            start_in(w + 1, 1 - slot)

        for dblk in range(D // 128):
            pltpu.async_copy(
                buf_slots[slot].at[:, pl.ds(dblk * 128, 128)],
                out_hbm.at[pl.ds(w * W, W), dblk, bsel, :],
                sem_out[slot])

    def pair_body(q, carry):
        w0 = lo + 2 * q
        for s in range(2):
            w = w0 + s

            @pl.when(w < hi)
            def _(w=w, s=s):
                do_window(w, s)

        return carry

    @pl.when(nt >= 1)
    def _():
        start_in(lo, 0)

    lax.fori_loop(0, (WPT + 1) // 2, pair_body, 0)

    @pl.when(nt >= 2)
    def _():
        wait_out(0)
        wait_out(1)

    @pl.when(nt == 1)
    def _():
        wait_out(0)

    @pl.when(t8 == 7)
    def _():
        fetch_ids(WPB * W, 0)
        pltpu.async_copy(x_b.at[idx0], buf0, sem_in[0]).wait()
        for dblk in range(D // 128):
            pltpu.sync_copy(
                buf0.at[pl.ds(0, TAIL), pl.ds(dblk * 128, 128)],
                out_hbm.at[pl.ds(WPB * W, TAIL), dblk, bsel, :])


def kernel(x, noise):
    b, n, d = x.shape
    assert (b, n, d) == (B, N, D)
    noise_bits = lax.bitcast_convert_type(noise.reshape(B * N), jnp.int32)
    out_kdb = _dropout_kernel(x, noise_bits)
    out_kdb = lax.optimization_barrier(out_kdb)
    return out_kdb.transpose(2, 0, 1, 3).reshape(B, K, D)


# speedup vs baseline: 1.9794x; 1.0401x over previous
import functools

import jax
import jax.numpy as jnp
from jax import lax
from jax.experimental import pallas as pl
from jax.experimental.pallas import tpu as pltpu
from jax.experimental.pallas import tpu_sc as plsc

B = 4
N = 4096
D = 2048
PROB_KEEP = 0.9
K = max(1, int(N * PROB_KEEP))
TOTAL = B * K
NC = 2
NS = 16
W = 16
WPB = K // W
TAIL = K - WPB * W
TPB = (NC * NS) // B
WPT = -(-WPB // TPB)

VREGS = N // 16

_mesh = plsc.VectorSubcoreMesh(
    core_axis_name="c", subcore_axis_name="s", num_cores=NC, num_subcores=NS
)


@functools.partial(
    pl.kernel,
    out_type=jax.ShapeDtypeStruct((K, D // 128, B, 128), jnp.float32),
    mesh=_mesh,
    compiler_params=pltpu.CompilerParams(needs_layout_passes=False),
    scratch_types=[
        pltpu.VMEM((N,), jnp.int32),
        pltpu.VMEM((16 * 257,), jnp.int32),
        pltpu.VMEM((N,), jnp.int32),
        pltpu.VMEM((N,), jnp.int32),
        pltpu.VMEM((N,), jnp.int32),
        pltpu.VMEM((N,), jnp.int32),
        pltpu.VMEM((N,), jnp.int32),
        pltpu.VMEM_SHARED((B * N,), jnp.int32),
        pltpu.VMEM((N,), jnp.int32),
        pltpu.VMEM((16,), jnp.int32),
        pltpu.VMEM((16,), jnp.int32),
        pltpu.VMEM((W, D), jnp.float32),
        pltpu.VMEM((W, D), jnp.float32),
        pltpu.SemaphoreType.DMA,
        pltpu.SemaphoreType.DMA,
        pltpu.SemaphoreType.DMA,
        pltpu.SemaphoreType.DMA,
    ],
)
def _dropout_kernel(x_hbm, noise_hbm, out_hbm,
                    noise_v, noise_pad, key_a, key_b, idx_a, idx_b, cnt,
                    tab_sh, table_v, idx0, idx1, buf0, buf1,
                    sem_in0, sem_in1, sem_out0, sem_out1):
    cid = lax.axis_index("c")
    sid = lax.axis_index("s")
    lane = lax.iota(jnp.int32, 16)

    @pl.when(sid < B)
    def _():
        bsel = sid
        pltpu.sync_copy(noise_hbm.at[pl.ds(bsel * N, N)], noise_v)
        ones = jnp.ones((16,), jnp.int32)

        def pad_body(v, carry):
            w = noise_v[pl.ds(v * 16, 16)]
            noise_pad[pl.ds((v >> 4) * 257 + (v & 15) * 16, 16)] = w
            return carry

        def pad4(q, c):
            for u in range(4):
                pad_body(4 * q + u, c)
            return c

        lax.fori_loop(0, VREGS // 4, pad4, 0)

        def init_body(j, carry):
            s = plsc.load_gather(noise_pad, [lane * 257 + j])
            s = jnp.where(s == jnp.int32(-2147483648), jnp.int32(0), s)
            negm = lax.shift_right_arithmetic(s, 31)
            key = s ^ (jnp.bitwise_not(negm) & jnp.int32(0x7FFFFFFF))
            key_a[pl.ds(j * 16, 16)] = key
            idx_a[pl.ds(j * 16, 16)] = j + (lane << 8)
            return carry

        def init4(q, c):
            for u in range(4):
                init_body(4 * q + u, c)
            return c

        lax.fori_loop(0, VREGS // 4, init4, 0)

        bufs = [(key_a, idx_a), (key_b, idx_b)]
        for p in range(4):
            src_k, src_i = bufs[p % 2]
            dst_k, dst_i = bufs[(p + 1) % 2]
            shift = 8 * p

            def zero_body(j, carry):
                cnt[pl.ds(j * 16, 16)] = jnp.zeros((16,), jnp.int32)
                return carry

            def zero_body4(q, c, f=zero_body):
                for u in range(4):
                    f(4 * q + u, c)
                return c

            lax.fori_loop(0, VREGS // 4, zero_body4, 0)

            def hist_body(j, carry, src_k=src_k, shift=shift):
                kv = src_k[pl.ds(j * 16, 16)]
                d = lax.shift_right_logical(kv, shift) & 255
                plsc.addupdate_scatter(cnt, [(d << 4) | lane], ones)
                return carry

            def hist_body4(q, c, f=hist_body):
                for u in range(4):
                    f(4 * q + u, c)
                return c

            lax.fori_loop(0, VREGS // 4, hist_body4, 0)

            def scan_body(j, carry):
                v = cnt[pl.ds(j * 16, 16)]
                cs = plsc.cumsum(v)
                cnt[pl.ds(j * 16, 16)] = cs - v + carry
                return carry + jnp.sum(v)

            lax.fori_loop(0, VREGS, scan_body, jnp.int32(0))

            last = p == 3

            def perm_body(j, carry, src_k=src_k, src_i=src_i, dst_k=dst_k,
                          dst_i=dst_i, shift=shift, last=last):
                kv = src_k[pl.ds(j * 16, 16)]
                iv = src_i[pl.ds(j * 16, 16)]
                d = lax.shift_right_logical(kv, shift) & 255
                c = (d << 4) | lane
                pos = plsc.load_gather(cnt, [c])
                plsc.store_scatter(cnt, [c], pos + 1)
                if last:
                    m2 = pos
                else:
                    m2 = ((pos & 255) << 4) | lax.shift_right_logical(pos, 8)
                    plsc.store_scatter(dst_k, [m2], kv)
                plsc.store_scatter(dst_i, [m2], iv)
                return carry

            def perm_body4(q, c, f=perm_body):
                for u in range(4):
                    f(4 * q + u, c)
                return c

            lax.fori_loop(0, VREGS // 4, perm_body4, 0)

        pltpu.sync_copy(idx_a, tab_sh.at[pl.ds(bsel * N, N)])

    plsc.subcore_barrier()

    gid = sid * NC + cid
    bsel = lax.shift_right_logical(gid, 3)
    t8 = gid & 7
    pltpu.sync_copy(tab_sh.at[pl.ds(bsel * N, N)], table_v)
    lo = t8 * WPT
    hi = jnp.minimum(lo + WPT, WPB)
    nt = hi - lo
    x_b = x_hbm.at[bsel]

    idx_slots = [idx0, idx1]
    buf_slots = [buf0, buf1]
    sem_in = [sem_in0, sem_in1]
    sem_out = [sem_out0, sem_out1]

    def fetch_ids(start, slot):
        g = plsc.load_gather(table_v, [start + lane])
        idx_slots[slot][...] = g

    def wait_out(slot):
        for dblk in range(D // 128):
            pltpu.make_async_copy(buf_slots[slot].at[:, pl.ds(0, 128)],
                                  out_hbm.at[pl.ds(0, W), 0, 0, :],
                                  sem_out[slot]).wait()

    def start_in(w, slot):
        fetch_ids(w * W, slot)
        pltpu.async_copy(x_b.at[idx_slots[slot]], buf_slots[slot],
                         sem_in[slot])

    def wait_in(slot):
        pltpu.make_async_copy(x_b.at[idx_slots[slot]], buf_slots[slot],
                              sem_in[slot]).wait()

    def do_window(w, slot):
        wait_in(slot)

        @pl.when(w + 1 < hi)
        def _():
            @pl.when(w - lo >= 1)
            def _():
                wait_out(1 - slot)

            start_in(w + 1, 1 - slot)

        for dblk in range(D // 128):
            pltpu.async_copy(
                buf_slots[slot].at[:, pl.ds(dblk * 128, 128)],
                out_hbm.at[pl.ds(w * W, W), dblk, bsel, :],
                sem_out[slot])

    def pair_body(q, carry):
        w0 = lo + 2 * q
        for s in range(2):
            w = w0 + s

            @pl.when(w < hi)
            def _(w=w, s=s):
                do_window(w, s)

        return carry

    @pl.when(nt >= 1)
    def _():
        start_in(lo, 0)

    lax.fori_loop(0, (WPT + 1) // 2, pair_body, 0)

    @pl.when(nt >= 2)
    def _():
        wait_out(0)
        wait_out(1)

    @pl.when(nt == 1)
    def _():
        wait_out(0)

    @pl.when(t8 == 7)
    def _():
        fetch_ids(WPB * W, 0)
        pltpu.async_copy(x_b.at[idx0], buf0, sem_in[0]).wait()
        for dblk in range(D // 128):
            pltpu.sync_copy(
                buf0.at[pl.ds(0, TAIL), pl.ds(dblk * 128, 128)],
                out_hbm.at[pl.ds(WPB * W, TAIL), dblk, bsel, :])


def kernel(x, noise):
    b, n, d = x.shape
    assert (b, n, d) == (B, N, D)
    noise_bits = lax.bitcast_convert_type(noise.reshape(B * N), jnp.int32)
    out_kdb = _dropout_kernel(x, noise_bits)
    out_kdb = lax.optimization_barrier(out_kdb)
    return out_kdb.transpose(2, 0, 1, 3).reshape(B, K, D)
